# baseline (speedup 1.0000x reference)
import jax
import jax.numpy as jnp
from jax import lax
from jax.experimental import pallas as pl
from jax.experimental.pallas import tpu as pltpu
from jax.experimental.pallas import tpu_sc as plsc

N_UID = 1000000
N_MID = 1000000
N_CAT = 1000
EMB = 16
B = 4096
L = 200
BETA = 0.9

NC = 2
NS = 16
NW = NC * NS
BPW = B // NW

SEG_A = 104
SEG_B = 96


def _seg_sum(ref, par, n):
    parts = [jnp.zeros((EMB,), jnp.float32) for _ in range(4)]
    for i in range(n):
        parts[i % 4] = parts[i % 4] + ref[par, i, :]
    return (parts[0] + parts[1]) + (parts[2] + parts[3])


def _din_sc_kernel(params_hbm, mid_b_hbm, cat_b_hbm,
                   c0_hbm, c1_hbm, mid_his_hbm, cat_his_hbm,
                   urows_hbm, mid_tab_hbm, cat_tab_hbm,
                   out_hbm,
                   params_v, midx_v, cidx_v,
                   cnt0_v, cnt1_v,
                   kfp_v, urows_v, mrows_v, crows_v,
                   mha_idx, mhb_idx, cha_idx, chb_idx,
                   mha_rows, mhb_rows, cha_rows, chb_rows,
                   out_v,
                   semi0, semi1, semg0, semg1):
    wid = lax.axis_index("c") * NS + lax.axis_index("s")
    base = wid * BPW
    semi = (semi0, semi1)
    semg = (semg0, semg1)

    pltpu.sync_copy(params_hbm, params_v)
    pltpu.sync_copy(mid_b_hbm.at[pl.ds(base, BPW)], midx_v)
    pltpu.sync_copy(cat_b_hbm.at[pl.ds(base, BPW)], cidx_v)
    pltpu.sync_copy(c0_hbm.at[pl.ds(base, BPW)], cnt0_v)
    pltpu.sync_copy(c1_hbm.at[pl.ds(base, BPW)], cnt1_v)

    pltpu.sync_copy(urows_hbm.at[pl.ds(base * EMB, BPW * EMB)], urows_v)
    cm = pltpu.async_copy(mid_tab_hbm.at[midx_v], mrows_v, semg0)
    cc = pltpu.async_copy(cat_tab_hbm.at[cidx_v], crows_v, semg0)
    cm.wait()
    cc.wait()

    lane = jax.lax.iota(jnp.int32, 16)
    f_acc_v = params_v[pl.ds(0, 16)]
    f_beta_v = params_v[pl.ds(16, 16)]
    f_om_v = params_v[pl.ds(32, 16)]
    bnd0_v = params_v[pl.ds(48, 16)]
    bnd1_v = params_v[pl.ds(64, 16)]

    def bn_body(k, _):
        k16 = pl.multiple_of(k * 16, 16)
        c0 = cnt0_v[pl.ds(k16, 16)]
        c1 = cnt1_v[pl.ds(k16, 16)]
        acc = f_acc_v * c0
        cur = f_beta_v * c0 + f_om_v * c1
        cmax = jnp.maximum(cur, acc)
        kf0 = jnp.where(cmax >= bnd0_v, 1.0, 0.0).astype(jnp.float32)
        kf1 = jnp.where(cmax >= bnd1_v, 1.0, 0.0).astype(jnp.float32)
        idx2 = (lane + k16) * 2
        plsc.store_scatter(kfp_v, [idx2], kf0)
        plsc.store_scatter(kfp_v, [idx2 + 1], kf1)
        return 0

    lax.fori_loop(0, BPW // 16, bn_body, 0)

    half = (lane >= 8).astype(jnp.int32)

    def single_body(k, _):
        k16 = pl.multiple_of(k * 16, 16)
        for j in range(16):
            i = k16 + j
            fidx = jnp.full((16,), 2 * i, jnp.int32) + half
            f_row = plsc.load_gather(kfp_v, [fidx])
            out_v[i, pl.ds(0, EMB)] = urows_v[pl.ds(i * EMB, EMB)] * f_row
            out_v[i, pl.ds(16, EMB)] = mrows_v[i, :]
            out_v[i, pl.ds(32, EMB)] = crows_v[i, :]
        return 0

    lax.fori_loop(0, BPW // 16, single_body, 0)

    def his_offs(lb):
        g = base + lb
        offa = pl.multiple_of(g * L, 8)
        offb = pl.multiple_of(g * L + SEG_A, 8)
        return offa, offb

    def fire_idx(lb, par):
        offa, offb = his_offs(lb)
        pltpu.async_copy(mid_his_hbm.at[pl.ds(offa, SEG_A)], mha_idx.at[par], semi[par])
        pltpu.async_copy(mid_his_hbm.at[pl.ds(offb, SEG_B)], mhb_idx.at[par], semi[par])
        pltpu.async_copy(cat_his_hbm.at[pl.ds(offa, SEG_A)], cha_idx.at[par], semi[par])
        pltpu.async_copy(cat_his_hbm.at[pl.ds(offb, SEG_B)], chb_idx.at[par], semi[par])

    def wait_idx(lb, par):
        offa, offb = his_offs(lb)
        pltpu.make_async_copy(mid_his_hbm.at[pl.ds(offa, SEG_A)], mha_idx.at[par], semi[par]).wait()
        pltpu.make_async_copy(mid_his_hbm.at[pl.ds(offb, SEG_B)], mhb_idx.at[par], semi[par]).wait()
        pltpu.make_async_copy(cat_his_hbm.at[pl.ds(offa, SEG_A)], cha_idx.at[par], semi[par]).wait()
        pltpu.make_async_copy(cat_his_hbm.at[pl.ds(offb, SEG_B)], chb_idx.at[par], semi[par]).wait()

    def fire_gather(par):
        pltpu.async_copy(mid_tab_hbm.at[mha_idx.at[par]], mha_rows.at[par], semg[par])
        pltpu.async_copy(mid_tab_hbm.at[mhb_idx.at[par]], mhb_rows.at[par], semg[par])
        pltpu.async_copy(cat_tab_hbm.at[cha_idx.at[par]], cha_rows.at[par], semg[par])
        pltpu.async_copy(cat_tab_hbm.at[chb_idx.at[par]], chb_rows.at[par], semg[par])

    def wait_gather(par):
        pltpu.make_async_copy(mid_tab_hbm.at[mha_idx.at[par]], mha_rows.at[par], semg[par]).wait()
        pltpu.make_async_copy(mid_tab_hbm.at[mhb_idx.at[par]], mhb_rows.at[par], semg[par]).wait()
        pltpu.make_async_copy(cat_tab_hbm.at[cha_idx.at[par]], cha_rows.at[par], semg[par]).wait()
        pltpu.make_async_copy(cat_tab_hbm.at[chb_idx.at[par]], chb_rows.at[par], semg[par]).wait()

    fire_idx(0, 0)
    fire_idx(1, 1)
    wait_idx(0, 0)
    fire_gather(0)

    def outer_body(g, _):
        for par in range(2):
            lb = g + par
            nxt = lb + 1

            @pl.when(nxt < BPW)
            def _():
                wait_idx(nxt, 1 - par)
                fire_gather(1 - par)

            wait_gather(par)

            @pl.when(lb + 2 < BPW)
            def _():
                fire_idx(lb + 2, par)

            acc_m = _seg_sum(mha_rows, par, SEG_A) + _seg_sum(mhb_rows, par, SEG_B)
            acc_c = _seg_sum(cha_rows, par, SEG_A) + _seg_sum(chb_rows, par, SEG_B)
            out_v[lb, pl.ds(48, EMB)] = acc_m
            out_v[lb, pl.ds(64, EMB)] = acc_c
        return 0

    lax.fori_loop(0, BPW // 2, lambda i, c: outer_body(i * 2, c), 0)

    pltpu.sync_copy(out_v, out_hbm.at[pl.ds(base, BPW), :])


@jax.jit
def _din_call(params, mid_b, cat_b, c0, c1, mid_his, cat_his,
              urows, mid_tab, cat_tab):
    mesh = plsc.VectorSubcoreMesh(core_axis_name="c", subcore_axis_name="s")
    f = pl.kernel(
        _din_sc_kernel,
        out_type=jax.ShapeDtypeStruct((B, 80), jnp.float32),
        mesh=mesh,
        compiler_params=pltpu.CompilerParams(
            use_tc_tiling_on_sc=False, needs_layout_passes=False),
        scratch_types=[
            pltpu.VMEM((80,), jnp.float32),
            pltpu.VMEM((BPW,), jnp.int32),
            pltpu.VMEM((BPW,), jnp.int32),
            pltpu.VMEM((BPW,), jnp.float32),
            pltpu.VMEM((BPW,), jnp.float32),
            pltpu.VMEM((2 * BPW,), jnp.float32),
            pltpu.VMEM((BPW * EMB,), jnp.float32),
            pltpu.VMEM((BPW, EMB), jnp.float32),
            pltpu.VMEM((BPW, EMB), jnp.float32),
            pltpu.VMEM((2, SEG_A), jnp.int32),
            pltpu.VMEM((2, SEG_B), jnp.int32),
            pltpu.VMEM((2, SEG_A), jnp.int32),
            pltpu.VMEM((2, SEG_B), jnp.int32),
            pltpu.VMEM((2, SEG_A, EMB), jnp.float32),
            pltpu.VMEM((2, SEG_B, EMB), jnp.float32),
            pltpu.VMEM((2, SEG_A, EMB), jnp.float32),
            pltpu.VMEM((2, SEG_B, EMB), jnp.float32),
            pltpu.VMEM((BPW, 80), jnp.float32),
            pltpu.SemaphoreType.DMA,
            pltpu.SemaphoreType.DMA,
            pltpu.SemaphoreType.DMA,
            pltpu.SemaphoreType.DMA,
        ],
    )
    return f(params, mid_b, cat_b, c0, c1, mid_his, cat_his,
             urows, mid_tab, cat_tab)


def kernel(uid_batch_ph, mid_batch_ph, cat_batch_ph, mid_his_batch_ph,
           cat_his_batch_ph, mask, seq_len_ph, uid_cnt, uid_bnd,
           cnt_reset_num, uid_table, mid_table, cat_table):
    num = cnt_reset_num[0]
    debias0 = jnp.maximum(jnp.abs(1.0 - BETA ** num), 1e-08)
    debias1 = jnp.abs(1.0 - BETA ** (num + 1.0))
    params = jnp.stack([
        1.0 / debias0,
        BETA / debias1,
        (1.0 - BETA) / debias1,
        uid_bnd[0],
        uid_bnd[1],
    ])
    params = jnp.repeat(params.astype(jnp.float32)[:, None], 16, axis=1).reshape(-1)
    cnt_rows = jnp.take(uid_cnt, uid_batch_ph, axis=0)
    big = jnp.float32(3.0e38)
    c0 = jnp.minimum(cnt_rows[:, 0], big)
    c1 = jnp.minimum(cnt_rows[:, 1], big)
    mh = mid_his_batch_ph.reshape(-1).astype(jnp.float32).astype(jnp.int32)
    ch = cat_his_batch_ph.reshape(-1).astype(jnp.float32).astype(jnp.int32)
    urows = jnp.minimum(jnp.take(uid_table, uid_batch_ph, axis=0),
                        big).reshape(-1)
    return _din_call(params, mid_batch_ph, cat_batch_ph, c0, c1,
                     mh, ch, urows, mid_table, cat_table)
